# baseline (speedup 1.0000x reference)
import functools

import jax
import jax.numpy as jnp
from jax import lax
from jax.experimental import pallas as pl
from jax.experimental.pallas import tpu as pltpu
from jax.experimental.pallas import tpu_sc as plsc

NUM_TYPES = 8
D = 1024
B = 8192
BM = 256
NBLK = B // BM
MAX_WORK = NBLK + NUM_TYPES - 1
SUB = 64
LANES = 128
NW = 32
ROWS_PER_W = B // NW
CHUNK = 32
NCH = ROWS_PER_W // CHUNK
NBUF = 3


def _routing_body(types_ref, pos_ref, starts_ref):
    bf = jnp.bfloat16
    t = types_ref[...]
    R = NUM_TYPES * SUB
    m_rows = [(t == tt).astype(bf) for tt in range(NUM_TYPES)]
    M = jnp.concatenate(m_rows, axis=0)

    r512 = lax.broadcasted_iota(jnp.int32, (R, R), 0)
    c512 = lax.broadcasted_iota(jnp.int32, (R, R), 1)
    sl512 = (c512 < r512).astype(bf)
    r128 = lax.broadcasted_iota(jnp.int32, (LANES, LANES), 0)
    c128 = lax.broadcasted_iota(jnp.int32, (LANES, LANES), 1)
    upper_incl = (r128 <= c128).astype(bf)
    ones_l = jnp.ones((LANES, LANES), dtype=bf)

    f32 = jnp.float32
    dot = functools.partial(lax.dot, preferred_element_type=f32)
    full_rows = dot(sl512, M)
    incl = dot(full_rows.astype(bf), ones_l) + dot(M, upper_incl)

    pos_f = jnp.zeros((SUB, LANES), dtype=f32)
    for tt in range(NUM_TYPES):
        blk = lax.slice_in_dim(incl, tt * SUB, (tt + 1) * SUB, axis=0)
        pos_f = pos_f + m_rows[tt].astype(f32) * (blk - 1.0)
    pos_ref[...] = pos_f.astype(jnp.int32)

    rsel = lax.broadcasted_iota(jnp.int32, (NUM_TYPES, R), 0)
    csel = lax.broadcasted_iota(jnp.int32, (NUM_TYPES, R), 1)
    sel = (csel // SUB == rsel).astype(bf)
    totals = dot(sel, M)
    totals = dot(totals.astype(bf), ones_l)
    start_rows = [jnp.zeros((1, LANES), dtype=f32)]
    acc = jnp.zeros((1, LANES), dtype=f32)
    for tt in range(1, NUM_TYPES):
        acc = acc + lax.slice_in_dim(totals, tt - 1, tt, axis=0)
        start_rows.append(acc)
    starts_ref[...] = jnp.concatenate(start_rows, axis=0).astype(jnp.int32)


def _routing(types2d):
    return pl.pallas_call(
        _routing_body,
        out_shape=(
            jax.ShapeDtypeStruct((SUB, LANES), jnp.int32),
            jax.ShapeDtypeStruct((NUM_TYPES, LANES), jnp.int32),
        ),
    )(types2d)


def _gmm_body(rb_ref, tb_ref, gs_ref, ge_ref, x_ref, w_ref, b_ref, out_ref):
    w = pl.program_id(0)
    rb = rb_ref[w]
    prev_rb = rb_ref[jnp.maximum(w - 1, 0)]
    is_first = jnp.logical_or(w == 0, rb != prev_rb)

    @pl.when(is_first)
    def _init():
        out_ref[...] = jnp.zeros_like(out_ref)

    gs = gs_ref[w]
    ge = ge_ref[w]

    @pl.when(gs < ge)
    def _compute():
        xb = x_ref[...].astype(jnp.bfloat16)
        wb = w_ref[0]
        acc = lax.dot_general(
            xb, wb, (((1,), (0,)), ((), ())),
            preferred_element_type=jnp.float32,
        )
        rows = rb * BM + lax.broadcasted_iota(jnp.int32, (BM, 1), 0)
        mask = jnp.logical_and(rows >= gs, rows < ge)
        out_ref[...] += jnp.where(mask, acc + b_ref[0], 0.0)


def _grouped_matmul(rb, tb, gs, ge, x_sorted, W, b):
    grid_spec = pltpu.PrefetchScalarGridSpec(
        num_scalar_prefetch=4,
        grid=(MAX_WORK,),
        in_specs=[
            pl.BlockSpec((BM, D), lambda w, rb, tb, gs, ge: (rb[w], 0)),
            pl.BlockSpec((1, D, D), lambda w, rb, tb, gs, ge: (tb[w], 0, 0)),
            pl.BlockSpec((1, 1, D), lambda w, rb, tb, gs, ge: (tb[w], 0, 0)),
        ],
        out_specs=pl.BlockSpec((BM, D), lambda w, rb, tb, gs, ge: (rb[w], 0)),
    )
    return pl.pallas_call(
        _gmm_body,
        grid_spec=grid_spec,
        out_shape=jax.ShapeDtypeStruct((B, D), jnp.float32),
        compiler_params=pltpu.CompilerParams(
            dimension_semantics=("arbitrary",),
        ),
    )(rb, tb, gs, ge, x_sorted,
      W.transpose(0, 2, 1).astype(jnp.bfloat16),
      b.reshape(NUM_TYPES, 1, D))


def _sc_scatter_body(x_hbm, pos_hbm, out_hbm, idx_v, rows, sems_l, sems_s):
    wid = lax.axis_index("s") * 2 + lax.axis_index("c")
    base = wid * ROWS_PER_W
    pltpu.sync_copy(pos_hbm.at[pl.ds(wid * NCH, NCH)], idx_v)
    d_load = [None] * NCH
    d_sc = [None] * NCH
    d_load[0] = pltpu.async_copy(
        x_hbm.at[pl.ds(base, CHUNK)], rows[0], sems_l.at[0])
    for k in range(NCH):
        d_load[k].wait()
        if k >= NBUF - 1:
            d_sc[k - (NBUF - 1)].wait()
        if k + 1 < NCH:
            nb = (k + 1) % NBUF
            d_load[k + 1] = pltpu.async_copy(
                x_hbm.at[pl.ds(base + (k + 1) * CHUNK, CHUNK)],
                rows[nb], sems_l.at[nb])
        d_sc[k] = pltpu.async_copy(
            rows[k % NBUF], out_hbm.at[idx_v.at[k]], sems_s.at[k % NBUF])
    for k in range(max(0, NCH - (NBUF - 1)), NCH):
        d_sc[k].wait()


def _sc_gather_body(y_hbm, pos_hbm, out_hbm, idx_v, rows, sems_g, sems_s):
    wid = lax.axis_index("s") * 2 + lax.axis_index("c")
    base = wid * ROWS_PER_W
    pltpu.sync_copy(pos_hbm.at[pl.ds(wid * NCH, NCH)], idx_v)
    d_g = [None] * NCH
    d_st = [None] * NCH
    d_g[0] = pltpu.async_copy(y_hbm.at[idx_v.at[0]], rows[0], sems_g.at[0])
    for k in range(NCH):
        d_g[k].wait()
        if k >= NBUF - 1:
            d_st[k - (NBUF - 1)].wait()
        if k + 1 < NCH:
            nb = (k + 1) % NBUF
            d_g[k + 1] = pltpu.async_copy(
                y_hbm.at[idx_v.at[k + 1]], rows[nb], sems_g.at[nb])
        d_st[k] = pltpu.async_copy(
            rows[k % NBUF], out_hbm.at[pl.ds(base + k * CHUNK, CHUNK)],
            sems_s.at[k % NBUF])
    for k in range(max(0, NCH - (NBUF - 1)), NCH):
        d_st[k].wait()


@functools.lru_cache(maxsize=None)
def _sc_kernels():
    mesh = plsc.VectorSubcoreMesh(
        core_axis_name="c", subcore_axis_name="s", num_cores=2, num_subcores=16
    )
    scratch = [
        pltpu.VMEM((NCH, CHUNK), jnp.int32),
        [pltpu.VMEM((CHUNK, D), jnp.float32) for _ in range(NBUF)],
        pltpu.SemaphoreType.DMA((NBUF,)),
        pltpu.SemaphoreType.DMA((NBUF,)),
    ]
    mk = functools.partial(
        pl.kernel,
        out_type=jax.ShapeDtypeStruct((B, D), jnp.float32),
        mesh=mesh,
        scratch_types=scratch,
    )
    return mk(_sc_scatter_body), mk(_sc_gather_body)


def _worklist(starts):
    i32 = jnp.int32
    ends = jnp.concatenate([starts[1:], jnp.array([B], dtype=i32)])
    counts = ends - starts
    nonempty = counts > 0
    first_blk = starts // BM
    last_blk = jnp.where(nonempty, (ends - 1) // BM, 0)
    n_items = jnp.where(nonempty, last_blk - first_blk + 1, 0)
    item_start = jnp.concatenate(
        [jnp.zeros((1,), dtype=i32), jnp.cumsum(n_items)[:-1].astype(i32)]
    )
    total = jnp.sum(n_items)
    wids = jnp.arange(MAX_WORK, dtype=i32)
    belongs = jnp.logical_and(
        wids[None, :] >= item_start[:, None],
        wids[None, :] < (item_start + n_items)[:, None],
    )
    g = jnp.argmax(belongs, axis=0).astype(i32)
    valid = wids < total
    g_last = jnp.argmax(
        jnp.where(nonempty, jnp.arange(NUM_TYPES, dtype=i32), -1)
    ).astype(i32)
    rb = jnp.where(valid, first_blk[g] + (wids - item_start[g]), NBLK - 1)
    tb = jnp.where(valid, g, g_last)
    gs = jnp.where(valid, starts[g], 0)
    ge = jnp.where(valid, ends[g], 0)
    return rb, tb, gs, ge


def kernel(x, types, W, b):
    types2d = types.reshape(SUB, LANES)
    pos2d, starts_rows = _routing(types2d)
    pos = pos2d.reshape(B)
    starts = starts_rows[:, 0]
    rb, tb, gs, ge = _worklist(starts)
    scatter_rows, gather_rows = _sc_kernels()
    pos_sc = pos.reshape(NW * NCH, CHUNK)
    x_sorted = scatter_rows(x, pos_sc)
    y_sorted = _grouped_matmul(rb, tb, gs, ge, x_sorted, W, b)
    return gather_rows(y_sorted, pos_sc)


# speedup vs baseline: 1.0548x; 1.0548x over previous
import functools

import jax
import jax.numpy as jnp
from jax import lax
from jax.experimental import pallas as pl
from jax.experimental.pallas import tpu as pltpu
from jax.experimental.pallas import tpu_sc as plsc

NUM_TYPES = 8
D = 1024
B = 8192
BM = 256
NBLK = B // BM
MAX_WORK = NBLK + NUM_TYPES - 1
SUB = 64
LANES = 128
NW = 32
ROWS_PER_W = B // NW
CHUNK = 32
NCH = ROWS_PER_W // CHUNK
NBUF = 3


def _routing_body(types_ref, pos_ref, wl_ref):
    bf = jnp.bfloat16
    t = types_ref[...]
    R = NUM_TYPES * SUB
    m_rows = [(t == tt).astype(bf) for tt in range(NUM_TYPES)]
    M = jnp.concatenate(m_rows, axis=0)

    r512 = lax.broadcasted_iota(jnp.int32, (R, R), 0)
    c512 = lax.broadcasted_iota(jnp.int32, (R, R), 1)
    sl512 = (c512 < r512).astype(bf)
    r128 = lax.broadcasted_iota(jnp.int32, (LANES, LANES), 0)
    c128 = lax.broadcasted_iota(jnp.int32, (LANES, LANES), 1)
    upper_incl = (r128 <= c128).astype(bf)
    ones_l = jnp.ones((LANES, LANES), dtype=bf)

    f32 = jnp.float32
    dot = functools.partial(lax.dot, preferred_element_type=f32)
    full_rows = dot(sl512, M)
    incl = dot(full_rows.astype(bf), ones_l) + dot(M, upper_incl)

    pos_f = jnp.zeros((SUB, LANES), dtype=f32)
    for tt in range(NUM_TYPES):
        blk = lax.slice_in_dim(incl, tt * SUB, (tt + 1) * SUB, axis=0)
        pos_f = pos_f + m_rows[tt].astype(f32) * (blk - 1.0)
    pos_ref[...] = pos_f.astype(jnp.int32)

    rsel = lax.broadcasted_iota(jnp.int32, (NUM_TYPES, R), 0)
    csel = lax.broadcasted_iota(jnp.int32, (NUM_TYPES, R), 1)
    sel = (csel // SUB == rsel).astype(bf)
    totals = dot(sel, M)
    totals = dot(totals.astype(bf), ones_l)
    start_rows = [jnp.zeros((1, LANES), dtype=f32)]
    acc = jnp.zeros((1, LANES), dtype=f32)
    for tt in range(1, NUM_TYPES):
        acc = acc + lax.slice_in_dim(totals, tt - 1, tt, axis=0)
        start_rows.append(acc)
    starts = jnp.concatenate(start_rows, axis=0)

    ends = jnp.concatenate(
        [lax.slice_in_dim(starts, 1, NUM_TYPES, axis=0),
         jnp.full((1, LANES), float(B), dtype=f32)], axis=0)
    counts = ends - starts
    nonempty = counts > 0.0
    first_blk = jnp.floor(starts * (1.0 / BM))
    last_blk = jnp.floor((ends - 1.0) * (1.0 / BM))
    n_items = jnp.where(nonempty, last_blk - first_blk + 1.0, 0.0)
    istart_rows = [jnp.zeros((1, LANES), dtype=f32)]
    acc2 = jnp.zeros((1, LANES), dtype=f32)
    for tt in range(1, NUM_TYPES):
        acc2 = acc2 + lax.slice_in_dim(n_items, tt - 1, tt, axis=0)
        istart_rows.append(acc2)
    item_start = jnp.concatenate(istart_rows, axis=0)
    total_items = jnp.sum(n_items, axis=0, keepdims=True)

    lmat = lax.broadcasted_iota(jnp.int32, (NUM_TYPES, LANES), 1).astype(f32)
    rowid = lax.broadcasted_iota(jnp.int32, (NUM_TYPES, LANES), 0).astype(f32)
    bel = jnp.logical_and(lmat >= item_start, lmat < item_start + n_items)

    def _sel(v):
        return jnp.sum(jnp.where(bel, v, 0.0), axis=0, keepdims=True)

    lane = lax.broadcasted_iota(jnp.int32, (1, LANES), 1).astype(f32)
    valid = lane < total_items
    g_last = jnp.max(jnp.where(nonempty, rowid, -1.0), axis=0, keepdims=True)
    rb = jnp.where(valid, _sel(first_blk) + (lane - _sel(item_start)),
                   float(NBLK - 1))
    tb = jnp.where(valid, _sel(rowid), g_last)
    gs = jnp.where(valid, _sel(starts), 0.0)
    ge = jnp.where(valid, _sel(ends), 0.0)
    pad = jnp.zeros((NUM_TYPES - 4, LANES), dtype=f32)
    wl_ref[...] = jnp.concatenate([rb, tb, gs, ge, pad], axis=0).astype(
        jnp.int32)


def _routing(types2d):
    return pl.pallas_call(
        _routing_body,
        out_shape=(
            jax.ShapeDtypeStruct((SUB, LANES), jnp.int32),
            jax.ShapeDtypeStruct((NUM_TYPES, LANES), jnp.int32),
        ),
    )(types2d)


def _gmm_body(wl_ref, x_ref, w_ref, b_ref, out_ref):
    w = pl.program_id(0)
    rb = wl_ref[0, w]
    prev_rb = wl_ref[0, jnp.maximum(w - 1, 0)]
    is_first = jnp.logical_or(w == 0, rb != prev_rb)
    gs = wl_ref[2, w]
    ge = wl_ref[3, w]
    rs = rb * BM
    interior = jnp.logical_and(gs <= rs, rs + BM <= ge)

    def _acc():
        xb = x_ref[...].astype(jnp.bfloat16)
        wb = w_ref[0]
        return lax.dot_general(
            xb, wb, (((1,), (0,)), ((), ())),
            preferred_element_type=jnp.float32,
        )

    @pl.when(interior)
    def _full_block():
        out_ref[...] = _acc() + b_ref[0]

    @pl.when(jnp.logical_and(jnp.logical_not(interior), gs < ge))
    def _boundary():
        @pl.when(is_first)
        def _init():
            out_ref[...] = jnp.zeros_like(out_ref)

        rows = rs + lax.broadcasted_iota(jnp.int32, (BM, 1), 0)
        mask = jnp.logical_and(rows >= gs, rows < ge)
        out_ref[...] += jnp.where(mask, _acc() + b_ref[0], 0.0)


def _grouped_matmul(wl, x_sorted, W, b):
    grid_spec = pltpu.PrefetchScalarGridSpec(
        num_scalar_prefetch=1,
        grid=(MAX_WORK,),
        in_specs=[
            pl.BlockSpec((BM, D), lambda w, wl: (wl[0, w], 0)),
            pl.BlockSpec((1, D, D), lambda w, wl: (wl[1, w], 0, 0)),
            pl.BlockSpec((1, 1, D), lambda w, wl: (wl[1, w], 0, 0)),
        ],
        out_specs=pl.BlockSpec((BM, D), lambda w, wl: (wl[0, w], 0)),
    )
    return pl.pallas_call(
        _gmm_body,
        grid_spec=grid_spec,
        out_shape=jax.ShapeDtypeStruct((B, D), jnp.float32),
        compiler_params=pltpu.CompilerParams(
            dimension_semantics=("arbitrary",),
        ),
    )(wl, x_sorted,
      W.transpose(0, 2, 1).astype(jnp.bfloat16),
      b.reshape(NUM_TYPES, 1, D))


def _sc_scatter_body(x_hbm, pos_hbm, out_hbm, idx_v, rows, sems_l, sems_s):
    wid = lax.axis_index("s") * 2 + lax.axis_index("c")
    base = wid * ROWS_PER_W
    pltpu.sync_copy(pos_hbm.at[pl.ds(wid * NCH, NCH)], idx_v)
    d_load = [None] * NCH
    d_sc = [None] * NCH
    d_load[0] = pltpu.async_copy(
        x_hbm.at[pl.ds(base, CHUNK)], rows[0], sems_l.at[0])
    for k in range(NCH):
        d_load[k].wait()
        if k >= NBUF - 1:
            d_sc[k - (NBUF - 1)].wait()
        if k + 1 < NCH:
            nb = (k + 1) % NBUF
            d_load[k + 1] = pltpu.async_copy(
                x_hbm.at[pl.ds(base + (k + 1) * CHUNK, CHUNK)],
                rows[nb], sems_l.at[nb])
        d_sc[k] = pltpu.async_copy(
            rows[k % NBUF], out_hbm.at[idx_v.at[k]], sems_s.at[k % NBUF])
    for k in range(max(0, NCH - (NBUF - 1)), NCH):
        d_sc[k].wait()


def _sc_gather_body(y_hbm, pos_hbm, out_hbm, idx_v, rows, sems_g, sems_s):
    wid = lax.axis_index("s") * 2 + lax.axis_index("c")
    base = wid * ROWS_PER_W
    pltpu.sync_copy(pos_hbm.at[pl.ds(wid * NCH, NCH)], idx_v)
    d_g = [None] * NCH
    d_st = [None] * NCH
    d_g[0] = pltpu.async_copy(y_hbm.at[idx_v.at[0]], rows[0], sems_g.at[0])
    for k in range(NCH):
        d_g[k].wait()
        if k >= NBUF - 1:
            d_st[k - (NBUF - 1)].wait()
        if k + 1 < NCH:
            nb = (k + 1) % NBUF
            d_g[k + 1] = pltpu.async_copy(
                y_hbm.at[idx_v.at[k + 1]], rows[nb], sems_g.at[nb])
        d_st[k] = pltpu.async_copy(
            rows[k % NBUF], out_hbm.at[pl.ds(base + k * CHUNK, CHUNK)],
            sems_s.at[k % NBUF])
    for k in range(max(0, NCH - (NBUF - 1)), NCH):
        d_st[k].wait()


@functools.lru_cache(maxsize=None)
def _sc_kernels():
    mesh = plsc.VectorSubcoreMesh(
        core_axis_name="c", subcore_axis_name="s", num_cores=2, num_subcores=16
    )
    scratch = [
        pltpu.VMEM((NCH, CHUNK), jnp.int32),
        [pltpu.VMEM((CHUNK, D), jnp.float32) for _ in range(NBUF)],
        pltpu.SemaphoreType.DMA((NBUF,)),
        pltpu.SemaphoreType.DMA((NBUF,)),
    ]
    mk = functools.partial(
        pl.kernel,
        out_type=jax.ShapeDtypeStruct((B, D), jnp.float32),
        mesh=mesh,
        scratch_types=scratch,
    )
    return mk(_sc_scatter_body), mk(_sc_gather_body)


def kernel(x, types, W, b):
    types2d = types.reshape(SUB, LANES)
    pos2d, wl = _routing(types2d)
    wlp = lax.slice(wl, (0, 0), (4, MAX_WORK))
    scatter_rows, gather_rows = _sc_kernels()
    pos_sc = pos2d.reshape(NW * NCH, CHUNK)
    x_sorted = scatter_rows(x, pos_sc)
    y_sorted = _grouped_matmul(wlp, x_sorted, W, b)
    return gather_rows(y_sorted, pos_sc)


# speedup vs baseline: 1.0732x; 1.0174x over previous
import functools

import jax
import jax.numpy as jnp
from jax import lax
from jax.experimental import pallas as pl
from jax.experimental.pallas import tpu as pltpu
from jax.experimental.pallas import tpu_sc as plsc

NUM_TYPES = 8
D = 1024
B = 8192
BM = 256
NBLK = B // BM
MAX_WORK = NBLK + NUM_TYPES - 1
SUB = 64
LANES = 128
NW = 32
ROWS_PER_W = B // NW
CHUNK = 64
NCH = ROWS_PER_W // CHUNK


def _routing_body(types_ref, pos_ref, wl_ref):
    bf = jnp.bfloat16
    t = types_ref[...]
    R = NUM_TYPES * SUB
    m_rows = [(t == tt).astype(bf) for tt in range(NUM_TYPES)]
    M = jnp.concatenate(m_rows, axis=0)

    r512 = lax.broadcasted_iota(jnp.int32, (R, R), 0)
    c512 = lax.broadcasted_iota(jnp.int32, (R, R), 1)
    sl512 = (c512 < r512).astype(bf)
    r128 = lax.broadcasted_iota(jnp.int32, (LANES, LANES), 0)
    c128 = lax.broadcasted_iota(jnp.int32, (LANES, LANES), 1)
    upper_incl = (r128 <= c128).astype(bf)
    ones_l = jnp.ones((LANES, LANES), dtype=bf)

    f32 = jnp.float32
    dot = functools.partial(lax.dot, preferred_element_type=f32)
    full_rows = dot(sl512, M)
    incl = dot(full_rows.astype(bf), ones_l) + dot(M, upper_incl)

    pos_f = jnp.zeros((SUB, LANES), dtype=f32)
    for tt in range(NUM_TYPES):
        blk = lax.slice_in_dim(incl, tt * SUB, (tt + 1) * SUB, axis=0)
        pos_f = pos_f + m_rows[tt].astype(f32) * (blk - 1.0)
    pos_ref[...] = pos_f.astype(jnp.int32)

    rsel = lax.broadcasted_iota(jnp.int32, (NUM_TYPES, R), 0)
    csel = lax.broadcasted_iota(jnp.int32, (NUM_TYPES, R), 1)
    sel = (csel // SUB == rsel).astype(bf)
    totals = dot(sel, M)
    totals = dot(totals.astype(bf), ones_l)
    start_rows = [jnp.zeros((1, LANES), dtype=f32)]
    acc = jnp.zeros((1, LANES), dtype=f32)
    for tt in range(1, NUM_TYPES):
        acc = acc + lax.slice_in_dim(totals, tt - 1, tt, axis=0)
        start_rows.append(acc)
    starts = jnp.concatenate(start_rows, axis=0)

    ends = jnp.concatenate(
        [lax.slice_in_dim(starts, 1, NUM_TYPES, axis=0),
         jnp.full((1, LANES), float(B), dtype=f32)], axis=0)
    counts = ends - starts
    nonempty = counts > 0.0
    first_blk = jnp.floor(starts * (1.0 / BM))
    last_blk = jnp.floor((ends - 1.0) * (1.0 / BM))
    n_items = jnp.where(nonempty, last_blk - first_blk + 1.0, 0.0)
    istart_rows = [jnp.zeros((1, LANES), dtype=f32)]
    acc2 = jnp.zeros((1, LANES), dtype=f32)
    for tt in range(1, NUM_TYPES):
        acc2 = acc2 + lax.slice_in_dim(n_items, tt - 1, tt, axis=0)
        istart_rows.append(acc2)
    item_start = jnp.concatenate(istart_rows, axis=0)
    total_items = jnp.sum(n_items, axis=0, keepdims=True)

    lmat = lax.broadcasted_iota(jnp.int32, (NUM_TYPES, LANES), 1).astype(f32)
    rowid = lax.broadcasted_iota(jnp.int32, (NUM_TYPES, LANES), 0).astype(f32)
    bel = jnp.logical_and(lmat >= item_start, lmat < item_start + n_items)

    def _sel(v):
        return jnp.sum(jnp.where(bel, v, 0.0), axis=0, keepdims=True)

    lane = lax.broadcasted_iota(jnp.int32, (1, LANES), 1).astype(f32)
    valid = lane < total_items
    g_last = jnp.max(jnp.where(nonempty, rowid, -1.0), axis=0, keepdims=True)
    rb = jnp.where(valid, _sel(first_blk) + (lane - _sel(item_start)),
                   float(NBLK - 1))
    tb = jnp.where(valid, _sel(rowid), g_last)
    gs = jnp.where(valid, _sel(starts), 0.0)
    ge = jnp.where(valid, _sel(ends), 0.0)
    pad = jnp.zeros((NUM_TYPES - 4, LANES), dtype=f32)
    wl_ref[...] = jnp.concatenate([rb, tb, gs, ge, pad], axis=0).astype(
        jnp.int32)


def _routing(types2d):
    return pl.pallas_call(
        _routing_body,
        out_shape=(
            jax.ShapeDtypeStruct((SUB, LANES), jnp.int32),
            jax.ShapeDtypeStruct((NUM_TYPES, LANES), jnp.int32),
        ),
    )(types2d)


def _gmm_body(wl_ref, x_ref, w_ref, b_ref, out_ref):
    w = pl.program_id(0)
    rb = wl_ref[0, w]
    prev_rb = wl_ref[0, jnp.maximum(w - 1, 0)]
    is_first = jnp.logical_or(w == 0, rb != prev_rb)
    gs = wl_ref[2, w]
    ge = wl_ref[3, w]
    rs = rb * BM
    interior = jnp.logical_and(gs <= rs, rs + BM <= ge)

    def _acc():
        xb = x_ref[...].astype(jnp.bfloat16)
        wb = w_ref[0]
        return lax.dot_general(
            xb, wb, (((1,), (0,)), ((), ())),
            preferred_element_type=jnp.float32,
        )

    @pl.when(interior)
    def _full_block():
        out_ref[...] = _acc() + b_ref[0]

    @pl.when(jnp.logical_and(jnp.logical_not(interior), gs < ge))
    def _boundary():
        @pl.when(is_first)
        def _init():
            out_ref[...] = jnp.zeros_like(out_ref)

        rows = rs + lax.broadcasted_iota(jnp.int32, (BM, 1), 0)
        mask = jnp.logical_and(rows >= gs, rows < ge)
        out_ref[...] += jnp.where(mask, _acc() + b_ref[0], 0.0)


def _grouped_matmul(wl, x_sorted, W, b):
    grid_spec = pltpu.PrefetchScalarGridSpec(
        num_scalar_prefetch=1,
        grid=(MAX_WORK,),
        in_specs=[
            pl.BlockSpec((BM, D), lambda w, wl: (wl[0, w], 0)),
            pl.BlockSpec((1, D, D), lambda w, wl: (wl[1, w], 0, 0)),
            pl.BlockSpec((1, 1, D), lambda w, wl: (wl[1, w], 0, 0)),
        ],
        out_specs=pl.BlockSpec((BM, D), lambda w, wl: (wl[0, w], 0)),
    )
    return pl.pallas_call(
        _gmm_body,
        grid_spec=grid_spec,
        out_shape=jax.ShapeDtypeStruct((B, D), jnp.float32),
        compiler_params=pltpu.CompilerParams(
            dimension_semantics=("arbitrary",),
        ),
    )(wl, x_sorted, _wprep(W), b.reshape(NUM_TYPES, 1, D))


def _wprep_body(w_ref, wt_ref):
    wt_ref[0] = w_ref[0].astype(jnp.bfloat16).T


def _wprep(W):
    return pl.pallas_call(
        _wprep_body,
        grid=(NUM_TYPES,),
        in_specs=[pl.BlockSpec((1, D, D), lambda t: (t, 0, 0))],
        out_specs=pl.BlockSpec((1, D, D), lambda t: (t, 0, 0)),
        out_shape=jax.ShapeDtypeStruct((NUM_TYPES, D, D), jnp.bfloat16),
    )(W)


def _sc_scatter_body(x_hbm, pos_hbm, out_hbm, idx_v, rows_v, sem):
    wid = lax.axis_index("s") * 2 + lax.axis_index("c")
    base = wid * ROWS_PER_W
    pltpu.sync_copy(pos_hbm.at[pl.ds(wid * NCH, NCH)], idx_v)
    for k in range(NCH):
        pltpu.sync_copy(x_hbm.at[pl.ds(base + k * CHUNK, CHUNK)], rows_v)
        pltpu.async_copy(rows_v, out_hbm.at[idx_v.at[k]], sem).wait()


def _sc_gather_body(y_hbm, pos_hbm, out_hbm, idx_v, rows_v, sem):
    wid = lax.axis_index("s") * 2 + lax.axis_index("c")
    base = wid * ROWS_PER_W
    pltpu.sync_copy(pos_hbm.at[pl.ds(wid * NCH, NCH)], idx_v)
    for k in range(NCH):
        pltpu.async_copy(y_hbm.at[idx_v.at[k]], rows_v, sem).wait()
        pltpu.sync_copy(rows_v, out_hbm.at[pl.ds(base + k * CHUNK, CHUNK)])


@functools.lru_cache(maxsize=None)
def _sc_kernels():
    mesh = plsc.VectorSubcoreMesh(
        core_axis_name="c", subcore_axis_name="s", num_cores=2, num_subcores=16
    )
    scratch = [
        pltpu.VMEM((NCH, CHUNK), jnp.int32),
        pltpu.VMEM((CHUNK, D), jnp.float32),
        pltpu.SemaphoreType.DMA,
    ]
    mk = functools.partial(
        pl.kernel,
        out_type=jax.ShapeDtypeStruct((B, D), jnp.float32),
        mesh=mesh,
        scratch_types=scratch,
    )
    return mk(_sc_scatter_body), mk(_sc_gather_body)


def kernel(x, types, W, b):
    types2d = types.reshape(SUB, LANES)
    pos2d, wl = _routing(types2d)
    wlp = lax.slice(wl, (0, 0), (4, MAX_WORK))
    scatter_rows, gather_rows = _sc_kernels()
    pos_sc = pos2d.reshape(NW * NCH, CHUNK)
    x_sorted = scatter_rows(x, pos_sc)
    y_sorted = _grouped_matmul(wlp, x_sorted, W, b)
    return gather_rows(y_sorted, pos_sc)


# speedup vs baseline: 1.1506x; 1.0720x over previous
import functools

import jax
import jax.numpy as jnp
from jax import lax
from jax.experimental import pallas as pl
from jax.experimental.pallas import tpu as pltpu
from jax.experimental.pallas import tpu_sc as plsc

NUM_TYPES = 8
D = 1024
B = 8192
BM = 512
NBLK = B // BM
MAX_WORK = NBLK + NUM_TYPES - 1
SUB = 64
LANES = 128
NW = 32
ROWS_PER_W = B // NW
CHUNK = 64
NCH = ROWS_PER_W // CHUNK


def _routing_body(types_ref, pos_ref, wl_ref):
    bf = jnp.bfloat16
    t = types_ref[...]
    R = NUM_TYPES * SUB
    m_rows = [(t == tt).astype(bf) for tt in range(NUM_TYPES)]
    M = jnp.concatenate(m_rows, axis=0)

    r512 = lax.broadcasted_iota(jnp.int32, (R, R), 0)
    c512 = lax.broadcasted_iota(jnp.int32, (R, R), 1)
    sl512 = (c512 < r512).astype(bf)
    r128 = lax.broadcasted_iota(jnp.int32, (LANES, LANES), 0)
    c128 = lax.broadcasted_iota(jnp.int32, (LANES, LANES), 1)
    upper_incl = (r128 <= c128).astype(bf)
    ones_l = jnp.ones((LANES, LANES), dtype=bf)

    f32 = jnp.float32
    dot = functools.partial(lax.dot, preferred_element_type=f32)
    full_rows = dot(sl512, M)
    incl = dot(full_rows.astype(bf), ones_l) + dot(M, upper_incl)

    pos_f = jnp.zeros((SUB, LANES), dtype=f32)
    for tt in range(NUM_TYPES):
        blk = lax.slice_in_dim(incl, tt * SUB, (tt + 1) * SUB, axis=0)
        pos_f = pos_f + m_rows[tt].astype(f32) * (blk - 1.0)
    pos_ref[...] = pos_f.astype(jnp.int32)

    rsel = lax.broadcasted_iota(jnp.int32, (NUM_TYPES, R), 0)
    csel = lax.broadcasted_iota(jnp.int32, (NUM_TYPES, R), 1)
    sel = (csel // SUB == rsel).astype(bf)
    totals = dot(sel, M)
    totals = dot(totals.astype(bf), ones_l)
    start_rows = [jnp.zeros((1, LANES), dtype=f32)]
    acc = jnp.zeros((1, LANES), dtype=f32)
    for tt in range(1, NUM_TYPES):
        acc = acc + lax.slice_in_dim(totals, tt - 1, tt, axis=0)
        start_rows.append(acc)
    starts = jnp.concatenate(start_rows, axis=0)

    ends = jnp.concatenate(
        [lax.slice_in_dim(starts, 1, NUM_TYPES, axis=0),
         jnp.full((1, LANES), float(B), dtype=f32)], axis=0)
    counts = ends - starts
    nonempty = counts > 0.0
    first_blk = jnp.floor(starts * (1.0 / BM))
    last_blk = jnp.floor((ends - 1.0) * (1.0 / BM))
    n_items = jnp.where(nonempty, last_blk - first_blk + 1.0, 0.0)
    istart_rows = [jnp.zeros((1, LANES), dtype=f32)]
    acc2 = jnp.zeros((1, LANES), dtype=f32)
    for tt in range(1, NUM_TYPES):
        acc2 = acc2 + lax.slice_in_dim(n_items, tt - 1, tt, axis=0)
        istart_rows.append(acc2)
    item_start = jnp.concatenate(istart_rows, axis=0)
    total_items = jnp.sum(n_items, axis=0, keepdims=True)

    lmat = lax.broadcasted_iota(jnp.int32, (NUM_TYPES, LANES), 1).astype(f32)
    rowid = lax.broadcasted_iota(jnp.int32, (NUM_TYPES, LANES), 0).astype(f32)
    bel = jnp.logical_and(lmat >= item_start, lmat < item_start + n_items)

    def _sel(v):
        return jnp.sum(jnp.where(bel, v, 0.0), axis=0, keepdims=True)

    lane = lax.broadcasted_iota(jnp.int32, (1, LANES), 1).astype(f32)
    valid = lane < total_items
    g_last = jnp.max(jnp.where(nonempty, rowid, -1.0), axis=0, keepdims=True)
    rb = jnp.where(valid, _sel(first_blk) + (lane - _sel(item_start)),
                   float(NBLK - 1))
    tb = jnp.where(valid, _sel(rowid), g_last)
    gs = jnp.where(valid, _sel(starts), 0.0)
    ge = jnp.where(valid, _sel(ends), 0.0)
    pad = jnp.zeros((NUM_TYPES - 4, LANES), dtype=f32)
    wl_ref[...] = jnp.concatenate([rb, tb, gs, ge, pad], axis=0).astype(
        jnp.int32)


def _routing(types2d):
    return pl.pallas_call(
        _routing_body,
        out_shape=(
            jax.ShapeDtypeStruct((SUB, LANES), jnp.int32),
            jax.ShapeDtypeStruct((NUM_TYPES, LANES), jnp.int32),
        ),
    )(types2d)


def _gmm_body(wl_ref, x_ref, w_ref, b_ref, out_ref):
    w = pl.program_id(0)
    rb = wl_ref[0, w]
    prev_rb = wl_ref[0, jnp.maximum(w - 1, 0)]
    is_first = jnp.logical_or(w == 0, rb != prev_rb)
    gs = wl_ref[2, w]
    ge = wl_ref[3, w]
    rs = rb * BM
    interior = jnp.logical_and(gs <= rs, rs + BM <= ge)

    def _acc():
        xb = x_ref[...].astype(jnp.bfloat16)
        wb = w_ref[0]
        return lax.dot_general(
            xb, wb, (((1,), (0,)), ((), ())),
            preferred_element_type=jnp.float32,
        )

    @pl.when(interior)
    def _full_block():
        out_ref[...] = _acc() + b_ref[0]

    @pl.when(jnp.logical_and(jnp.logical_not(interior), gs < ge))
    def _boundary():
        @pl.when(is_first)
        def _init():
            out_ref[...] = jnp.zeros_like(out_ref)

        rows = rs + lax.broadcasted_iota(jnp.int32, (BM, 1), 0)
        mask = jnp.logical_and(rows >= gs, rows < ge)
        out_ref[...] += jnp.where(mask, _acc() + b_ref[0], 0.0)


def _grouped_matmul(wl, x_sorted, W, b):
    grid_spec = pltpu.PrefetchScalarGridSpec(
        num_scalar_prefetch=1,
        grid=(MAX_WORK,),
        in_specs=[
            pl.BlockSpec((BM, D), lambda w, wl: (wl[0, w], 0)),
            pl.BlockSpec((1, D, D), lambda w, wl: (wl[1, w], 0, 0)),
            pl.BlockSpec((1, 1, D), lambda w, wl: (wl[1, w], 0, 0)),
        ],
        out_specs=pl.BlockSpec((BM, D), lambda w, wl: (wl[0, w], 0)),
    )
    return pl.pallas_call(
        _gmm_body,
        grid_spec=grid_spec,
        out_shape=jax.ShapeDtypeStruct((B, D), jnp.float32),
        compiler_params=pltpu.CompilerParams(
            dimension_semantics=("arbitrary",),
        ),
    )(wl, x_sorted, _wprep(W), b.reshape(NUM_TYPES, 1, D))


def _wprep_body(w_ref, wt_ref):
    wt_ref[0] = w_ref[0].astype(jnp.bfloat16).T


def _wprep(W):
    return pl.pallas_call(
        _wprep_body,
        grid=(NUM_TYPES,),
        in_specs=[pl.BlockSpec((1, D, D), lambda t: (t, 0, 0))],
        out_specs=pl.BlockSpec((1, D, D), lambda t: (t, 0, 0)),
        out_shape=jax.ShapeDtypeStruct((NUM_TYPES, D, D), jnp.bfloat16),
    )(W)


def _sc_scatter_body(x_hbm, pos_hbm, out_hbm, idx_v, rows_v, sem):
    wid = lax.axis_index("s") * 2 + lax.axis_index("c")
    base = wid * ROWS_PER_W
    for k in range(NCH):
        off = base + k * CHUNK
        pltpu.sync_copy(pos_hbm.at[pl.ds(off, CHUNK)], idx_v)
        pltpu.sync_copy(x_hbm.at[pl.ds(off, CHUNK)], rows_v)
        pltpu.async_copy(rows_v, out_hbm.at[idx_v], sem).wait()


def _sc_gather_body(y_hbm, pos_hbm, out_hbm, idx_v, rows_v, sem):
    wid = lax.axis_index("s") * 2 + lax.axis_index("c")
    base = wid * ROWS_PER_W
    for k in range(NCH):
        off = base + k * CHUNK
        pltpu.sync_copy(pos_hbm.at[pl.ds(off, CHUNK)], idx_v)
        pltpu.async_copy(y_hbm.at[idx_v], rows_v, sem).wait()
        pltpu.sync_copy(rows_v, out_hbm.at[pl.ds(off, CHUNK)])


@functools.lru_cache(maxsize=None)
def _sc_kernels():
    mesh = plsc.VectorSubcoreMesh(
        core_axis_name="c", subcore_axis_name="s", num_cores=2, num_subcores=16
    )
    scratch = [
        pltpu.VMEM((CHUNK,), jnp.int32),
        pltpu.VMEM((CHUNK, D), jnp.float32),
        pltpu.SemaphoreType.DMA,
    ]
    mk = functools.partial(
        pl.kernel,
        out_type=jax.ShapeDtypeStruct((B, D), jnp.float32),
        mesh=mesh,
        scratch_types=scratch,
    )
    return mk(_sc_scatter_body), mk(_sc_gather_body)


def kernel(x, types, W, b):
    types2d = types.reshape(SUB, LANES)
    pos2d, wl = _routing(types2d)
    wlp = lax.slice(wl, (0, 0), (4, MAX_WORK))
    scatter_rows, gather_rows = _sc_kernels()
    pos_sc = pos2d.reshape(B)
    x_sorted = scatter_rows(x, pos_sc)
    y_sorted = _grouped_matmul(wlp, x_sorted, W, b)
    return gather_rows(y_sorted, pos_sc)


# speedup vs baseline: 1.2362x; 1.0744x over previous
import functools

import jax
import jax.numpy as jnp
from jax import lax
from jax.experimental import pallas as pl
from jax.experimental.pallas import tpu as pltpu
from jax.experimental.pallas import tpu_sc as plsc

NUM_TYPES = 8
D = 1024
B = 8192
BM = 512
NBLK = B // BM
MAX_WORK = NBLK + NUM_TYPES - 1
SUB = 64
LANES = 128
NW = 32
ROWS_PER_W = B // NW
CHUNK = 64
NCH = ROWS_PER_W // CHUNK


def _routing_body(types_ref, pos_ref, wl_ref):
    bf = jnp.bfloat16
    t = types_ref[...]
    R = NUM_TYPES * SUB
    m_rows = [(t == tt).astype(bf) for tt in range(NUM_TYPES)]
    M = jnp.concatenate(m_rows, axis=0)

    r512 = lax.broadcasted_iota(jnp.int32, (R, R), 0)
    c512 = lax.broadcasted_iota(jnp.int32, (R, R), 1)
    sl512 = (c512 < r512).astype(bf)
    r128 = lax.broadcasted_iota(jnp.int32, (LANES, LANES), 0)
    c128 = lax.broadcasted_iota(jnp.int32, (LANES, LANES), 1)
    upper_incl = (r128 <= c128).astype(bf)
    ones_l = jnp.ones((LANES, LANES), dtype=bf)

    f32 = jnp.float32
    dot = functools.partial(lax.dot, preferred_element_type=f32)
    full_rows = dot(sl512, M)
    incl = dot(full_rows.astype(bf), ones_l) + dot(M, upper_incl)

    pos_f = jnp.zeros((SUB, LANES), dtype=f32)
    for tt in range(NUM_TYPES):
        blk = lax.slice_in_dim(incl, tt * SUB, (tt + 1) * SUB, axis=0)
        pos_f = pos_f + m_rows[tt].astype(f32) * (blk - 1.0)
    pos_ref[...] = pos_f.astype(jnp.int32)

    rsel = lax.broadcasted_iota(jnp.int32, (NUM_TYPES, R), 0)
    csel = lax.broadcasted_iota(jnp.int32, (NUM_TYPES, R), 1)
    sel = (csel // SUB == rsel).astype(bf)
    totals = dot(sel, M)
    totals = dot(totals.astype(bf), ones_l)
    start_rows = [jnp.zeros((1, LANES), dtype=f32)]
    acc = jnp.zeros((1, LANES), dtype=f32)
    for tt in range(1, NUM_TYPES):
        acc = acc + lax.slice_in_dim(totals, tt - 1, tt, axis=0)
        start_rows.append(acc)
    starts = jnp.concatenate(start_rows, axis=0)

    ends = jnp.concatenate(
        [lax.slice_in_dim(starts, 1, NUM_TYPES, axis=0),
         jnp.full((1, LANES), float(B), dtype=f32)], axis=0)
    counts = ends - starts
    nonempty = counts > 0.0
    first_blk = jnp.floor(starts * (1.0 / BM))
    last_blk = jnp.floor((ends - 1.0) * (1.0 / BM))
    n_items = jnp.where(nonempty, last_blk - first_blk + 1.0, 0.0)
    istart_rows = [jnp.zeros((1, LANES), dtype=f32)]
    acc2 = jnp.zeros((1, LANES), dtype=f32)
    for tt in range(1, NUM_TYPES):
        acc2 = acc2 + lax.slice_in_dim(n_items, tt - 1, tt, axis=0)
        istart_rows.append(acc2)
    item_start = jnp.concatenate(istart_rows, axis=0)
    total_items = jnp.sum(n_items, axis=0, keepdims=True)

    lmat = lax.broadcasted_iota(jnp.int32, (NUM_TYPES, LANES), 1).astype(f32)
    rowid = lax.broadcasted_iota(jnp.int32, (NUM_TYPES, LANES), 0).astype(f32)
    bel = jnp.logical_and(lmat >= item_start, lmat < item_start + n_items)

    def _sel(v):
        return jnp.sum(jnp.where(bel, v, 0.0), axis=0, keepdims=True)

    lane = lax.broadcasted_iota(jnp.int32, (1, LANES), 1).astype(f32)
    valid = lane < total_items
    g_last = jnp.max(jnp.where(nonempty, rowid, -1.0), axis=0, keepdims=True)
    rb = jnp.where(valid, _sel(first_blk) + (lane - _sel(item_start)),
                   float(NBLK - 1))
    tb = jnp.where(valid, _sel(rowid), g_last)
    gs = jnp.where(valid, _sel(starts), 0.0)
    ge = jnp.where(valid, _sel(ends), 0.0)
    pad = jnp.zeros((NUM_TYPES - 4, LANES), dtype=f32)
    wl_ref[...] = jnp.concatenate([rb, tb, gs, ge, pad], axis=0).astype(
        jnp.int32)


def _routing(types2d):
    return pl.pallas_call(
        _routing_body,
        out_shape=(
            jax.ShapeDtypeStruct((SUB, LANES), jnp.int32),
            jax.ShapeDtypeStruct((NUM_TYPES, LANES), jnp.int32),
        ),
    )(types2d)


def _gmm_body(wl_ref, x_ref, w_ref, b_ref, out_ref):
    w = pl.program_id(0)
    rb = wl_ref[0, w]
    prev_rb = wl_ref[0, jnp.maximum(w - 1, 0)]
    is_first = jnp.logical_or(w == 0, rb != prev_rb)
    gs = wl_ref[2, w]
    ge = wl_ref[3, w]
    rs = rb * BM
    interior = jnp.logical_and(gs <= rs, rs + BM <= ge)

    def _acc():
        xb = x_ref[...].astype(jnp.bfloat16)
        wb = w_ref[0].astype(jnp.bfloat16)
        return lax.dot_general(
            xb, wb, (((1,), (1,)), ((), ())),
            preferred_element_type=jnp.float32,
        )

    @pl.when(interior)
    def _full_block():
        out_ref[...] = _acc() + b_ref[0]

    @pl.when(jnp.logical_and(jnp.logical_not(interior), gs < ge))
    def _boundary():
        @pl.when(is_first)
        def _init():
            out_ref[...] = jnp.zeros_like(out_ref)

        rows = rs + lax.broadcasted_iota(jnp.int32, (BM, 1), 0)
        mask = jnp.logical_and(rows >= gs, rows < ge)
        out_ref[...] += jnp.where(mask, _acc() + b_ref[0], 0.0)


def _grouped_matmul(wl, x_sorted, W, b):
    grid_spec = pltpu.PrefetchScalarGridSpec(
        num_scalar_prefetch=1,
        grid=(MAX_WORK,),
        in_specs=[
            pl.BlockSpec((BM, D), lambda w, wl: (wl[0, w], 0)),
            pl.BlockSpec((1, D, D), lambda w, wl: (wl[1, w], 0, 0)),
            pl.BlockSpec((1, 1, D), lambda w, wl: (wl[1, w], 0, 0)),
        ],
        out_specs=pl.BlockSpec((BM, D), lambda w, wl: (wl[0, w], 0)),
    )
    return pl.pallas_call(
        _gmm_body,
        grid_spec=grid_spec,
        out_shape=jax.ShapeDtypeStruct((B, D), jnp.float32),
        compiler_params=pltpu.CompilerParams(
            dimension_semantics=("arbitrary",),
        ),
    )(wl, x_sorted, W, b.reshape(NUM_TYPES, 1, D))


def _sc_scatter_body(x_hbm, pos_hbm, out_hbm, idx_v, rows_v, sem):
    wid = lax.axis_index("s") * 2 + lax.axis_index("c")
    base = wid * ROWS_PER_W
    for k in range(NCH):
        off = base + k * CHUNK
        pltpu.sync_copy(pos_hbm.at[pl.ds(off, CHUNK)], idx_v)
        pltpu.sync_copy(x_hbm.at[pl.ds(off, CHUNK)], rows_v)
        pltpu.async_copy(rows_v, out_hbm.at[idx_v], sem).wait()


def _sc_gather_body(y_hbm, pos_hbm, out_hbm, idx_v, rows_v, sem):
    wid = lax.axis_index("s") * 2 + lax.axis_index("c")
    base = wid * ROWS_PER_W
    for k in range(NCH):
        off = base + k * CHUNK
        pltpu.sync_copy(pos_hbm.at[pl.ds(off, CHUNK)], idx_v)
        pltpu.async_copy(y_hbm.at[idx_v], rows_v, sem).wait()
        pltpu.sync_copy(rows_v, out_hbm.at[pl.ds(off, CHUNK)])


@functools.lru_cache(maxsize=None)
def _sc_kernels():
    mesh = plsc.VectorSubcoreMesh(
        core_axis_name="c", subcore_axis_name="s", num_cores=2, num_subcores=16
    )
    scratch = [
        pltpu.VMEM((CHUNK,), jnp.int32),
        pltpu.VMEM((CHUNK, D), jnp.float32),
        pltpu.SemaphoreType.DMA,
    ]
    mk = functools.partial(
        pl.kernel,
        out_type=jax.ShapeDtypeStruct((B, D), jnp.float32),
        mesh=mesh,
        scratch_types=scratch,
    )
    return mk(_sc_scatter_body), mk(_sc_gather_body)


def kernel(x, types, W, b):
    types2d = types.reshape(SUB, LANES)
    pos2d, wl = _routing(types2d)
    wlp = lax.slice(wl, (0, 0), (4, MAX_WORK))
    scatter_rows, gather_rows = _sc_kernels()
    pos_sc = pos2d.reshape(B)
    x_sorted = scatter_rows(x, pos_sc)
    y_sorted = _grouped_matmul(wlp, x_sorted, W, b)
    return gather_rows(y_sorted, pos_sc)
